# baseline (speedup 1.0000x reference)
import functools

import jax
import jax.numpy as jnp
from jax import lax
from jax.experimental import pallas as pl
from jax.experimental.pallas import tpu as pltpu
from jax.experimental.pallas import tpu_sc as plsc

BATCH = 16384
L = 16
NI = 100
NC = 2
NS = 16
NW = NC * NS
BPW = BATCH // NW
CH = 128
NCH = BPW // CH

TBLK = 2048
NTB = BATCH // TBLK


def _sc_gather(users, ue_tabs, us_tabs):
    mesh = plsc.VectorSubcoreMesh(core_axis_name="c", subcore_axis_name="s")
    out_type = ([jax.ShapeDtypeStruct((BATCH, L), jnp.float32)] * 5
                + [jax.ShapeDtypeStruct((BATCH,), jnp.float32)] * 6)
    scratch = ([pltpu.VMEM((NCH, CH), jnp.int32)]
               + [pltpu.VMEM((BPW, L), jnp.float32)] * 5
               + [pltpu.VMEM((BPW,), jnp.float32)] * 6
               + [pltpu.SemaphoreType.DMA])

    @functools.partial(pl.kernel, mesh=mesh, out_type=out_type,
                       scratch_types=scratch,
                       compiler_params=pltpu.CompilerParams(
                           use_tc_tiling_on_sc=False))
    def k(users_hbm, *refs):
        tabs = refs[0:11]
        outs = refs[11:22]
        idx_v = refs[22]
        bufs = refs[23:34]
        sem = refs[34]
        wid = lax.axis_index("s") * NC + lax.axis_index("c")
        base = wid * BPW
        for j in range(NCH):
            pltpu.sync_copy(users_hbm.at[pl.ds(base + j * CH, CH)],
                            idx_v.at[j])
        cps = []
        for t in range(11):
            for j in range(NCH):
                if t < 5:
                    dst = bufs[t].at[pl.ds(j * CH, CH), :]
                else:
                    dst = bufs[t].at[pl.ds(j * CH, CH)]
                cps.append(pltpu.async_copy(tabs[t].at[idx_v.at[j]], dst, sem))
        for cp in cps:
            cp.wait()
        for t in range(11):
            if t < 5:
                pltpu.sync_copy(bufs[t], outs[t].at[pl.ds(base, BPW), :])
            else:
                pltpu.sync_copy(bufs[t], outs[t].at[pl.ds(base, BPW)])

    return k(users, *ue_tabs, *us_tabs)


def _tc_body(items_ref, gea, geb, gel, geg, ged,
             uba, ubb, ubl, ubg, ubd, refr,
             dist_ref, price_ref,
             iba, ibb, ibl, ibg, ibd,
             iea, ieb, iel, ieg, ied,
             gba, gbb, gbl, gbg, gbd,
             out_ref):
    it = items_ref[...]
    onehot = (it == lax.broadcasted_iota(jnp.int32, (TBLK, NI), 1)
              ).astype(jnp.float32)

    def ig(r):
        return jnp.dot(onehot, r[...], preferred_element_type=jnp.float32)

    def coef(gb, ub, ib, ue, ie):
        d = jnp.sum(ue[...] * ig(ie), axis=1, keepdims=True)
        return gb[0, 0] + ub[...] + ig(ib) + d

    alpha = coef(gba, uba, iba, gea, iea)
    beta = coef(gbb, ubb, ibb, geb, ieb)
    lamda = coef(gbl, ubl, ibl, gel, iel)
    gamma = coef(gbg, ubg, ibg, geg, ieg)
    delta = coef(gbd, ubd, ibd, ged, ied)

    dist = ig(dist_ref)
    price = ig(price_ref)

    rating = lax.broadcasted_iota(jnp.int32, (TBLK, 5), 1).astype(jnp.float32) + 1.0
    x = jnp.tanh(rating - refr[...])
    x_pos = (x > 0).astype(jnp.float32)
    x_neg = 1.0 - x_pos
    x_ = price * jnp.abs(x)
    v = x_ ** (alpha * x_pos + beta * x_neg)
    value = v * (x_pos - lamda * x_neg)
    w_exp = x_pos * gamma + x_neg * delta
    w_nom = dist ** w_exp
    w_den = (w_nom + (1.0 - dist) ** w_exp) ** (1.0 / w_exp)
    out_ref[...] = jnp.sum((w_nom / w_den) * value, axis=1, keepdims=True)


def _tc_math(items, ge, gus, p):
    col = lambda a: pl.BlockSpec((TBLK, 1), lambda i: (i, 0))
    full = lambda a: pl.BlockSpec(a.shape, lambda i: (0,) * a.ndim)
    emb = pl.BlockSpec((TBLK, L), lambda i: (i, 0))

    items2 = items.reshape(BATCH, 1)
    gus2 = [g.reshape(BATCH, 1) for g in gus]
    dist = p["dist"]
    price2 = p["price"].reshape(NI, 1)
    ibs = [p["ib_" + t] for t in ("a", "b", "l", "g", "d")]
    ies = [p["ie_" + t] for t in ("a", "b", "l", "g", "d")]
    gbs = [p["gb_" + t] for t in ("a", "b", "l", "g", "d")]

    args = [items2] + list(ge) + gus2 + [dist, price2] + ibs + ies + gbs
    specs = ([col(items2)] + [emb] * 5 + [col(g) for g in gus2]
             + [full(dist), full(price2)]
             + [full(a) for a in ibs] + [full(a) for a in ies]
             + [full(a) for a in gbs])
    out = pl.pallas_call(
        _tc_body,
        grid=(NTB,),
        in_specs=specs,
        out_specs=pl.BlockSpec((TBLK, 1), lambda i: (i, 0)),
        out_shape=jax.ShapeDtypeStruct((BATCH, 1), jnp.float32),
    )(*args)
    return out.reshape(BATCH)


def kernel(params, users, items):
    p = params
    U = p["ref"].shape[0]
    ue_tabs = [p["ue_" + t] for t in ("a", "b", "l", "g", "d")]
    us_tabs = [p["ub_" + t].reshape(U) for t in ("a", "b", "l", "g", "d")]
    us_tabs.append(p["ref"].reshape(U))
    g = _sc_gather(users, ue_tabs, us_tabs)
    ge, gus = g[0:5], g[5:11]
    return _tc_math(items, ge, gus, p)


# speedup vs baseline: 1.1543x; 1.1543x over previous
import functools

import jax
import jax.numpy as jnp
from jax import lax
from jax.experimental import pallas as pl
from jax.experimental.pallas import tpu as pltpu
from jax.experimental.pallas import tpu_sc as plsc

BATCH = 16384
L = 16
NI = 100
NC = 2
NS = 16
NW = NC * NS
BPW = BATCH // NW
CH = 128
NCH = BPW // CH

RB = 10240
TBLK = 2048
NTB = BATCH // TBLK


def _relayout_body(*refs):
    eye = (lax.broadcasted_iota(jnp.int32, (L, L), 0)
           == lax.broadcasted_iota(jnp.int32, (L, L), 1)).astype(jnp.float32)
    for i in range(5):
        x = refs[i][...]
        xt = jnp.dot(jnp.transpose(x), eye,
                     preferred_element_type=jnp.float32)
        y = xt.reshape(RB // 8, 8, L)
        for v in range(8):
            refs[5 + i][:, L * v:L * (v + 1)] = y[:, v, :]


def _relayout(ue_tabs):
    U = ue_tabs[0].shape[0]
    n = (U + RB - 1) // RB
    ins = [t.T for t in ue_tabs]
    in_specs = [pl.BlockSpec((L, RB), lambda i: (0, i))] * 5
    out_specs = [pl.BlockSpec((RB // 8, 128), lambda i: (i, 0))] * 5
    outs = pl.pallas_call(
        _relayout_body,
        grid=(n,),
        in_specs=in_specs,
        out_specs=out_specs,
        out_shape=[jax.ShapeDtypeStruct((U // 8, 128), jnp.float32)] * 5,
        compiler_params=pltpu.CompilerParams(
            fuse_transposed_lhs_in_matmul=True),
    )(*ins)
    return [o.reshape(U * L) for o in outs]


def _sc_gather(users, lin_tabs, us_tabs):
    mesh = plsc.VectorSubcoreMesh(core_axis_name="c", subcore_axis_name="s")
    out_type = ([jax.ShapeDtypeStruct((BATCH // 8, 128), jnp.float32)] * 5
                + [jax.ShapeDtypeStruct((BATCH,), jnp.float32)] * 6)
    scratch = ([pltpu.VMEM((NCH, CH), jnp.int32)]
               + [pltpu.VMEM((L, CH), jnp.int32)]
               + [pltpu.VMEM((L, CH), jnp.float32)] * 5
               + [pltpu.VMEM((CH,), jnp.float32)] * 6
               + [pltpu.SemaphoreType.DMA] * 2)

    @functools.partial(pl.kernel, mesh=mesh, out_type=out_type,
                       scratch_types=scratch,
                       compiler_params=pltpu.CompilerParams(
                           use_tc_tiling_on_sc=False))
    def k(users_hbm, *refs):
        tabs = refs[0:5]
        stabs = refs[5:11]
        outs = refs[11:16]
        souts = refs[16:22]
        idx_v = refs[22]
        lists = refs[23]
        bufs = refs[24:29]
        sbufs = refs[29:35]
        sem, sem2 = refs[35], refs[36]
        wid = lax.axis_index("s") * NC + lax.axis_index("c")
        base = wid * BPW
        for c in range(NCH):
            pltpu.sync_copy(users_hbm.at[pl.ds(base + c * CH, CH)],
                            idx_v.at[c])
        iota = lax.iota(jnp.int32, 16)
        for c in range(NCH):
            scps = [pltpu.async_copy(stabs[t].at[idx_v.at[c]], sbufs[t], sem2)
                    for t in range(6)]
            for m in range(8):
                vec = idx_v[c, pl.ds(16 * m, 16)]
                for half in range(2):
                    k_ = 2 * m + half
                    for v in range(8):
                        sel = jnp.full((16,), 8 * half + v, jnp.int32)
                        u16 = vec.at[sel].get(mode="promise_in_bounds")
                        lists[k_, pl.ds(16 * v, 16)] = u16 * L + iota
            cps = []
            for t in range(5):
                for k_ in range(L):
                    cps.append(pltpu.async_copy(
                        tabs[t].at[lists.at[k_]], bufs[t].at[k_], sem))
            for cp in cps:
                cp.wait()
            row0 = (base + c * CH) // 8
            for t in range(5):
                pltpu.sync_copy(bufs[t], outs[t].at[pl.ds(row0, L), :])
            for cp in scps:
                cp.wait()
            for t in range(6):
                pltpu.sync_copy(sbufs[t],
                                souts[t].at[pl.ds(base + c * CH, CH)])

    return k(users, *lin_tabs, *us_tabs)


def _tc_body(items_ref, gea, geb, gel, geg, ged,
             uba, ubb, ubl, ubg, ubd, refr,
             dist_ref, price_ref,
             iba, ibb, ibl, ibg, ibd,
             iea, ieb, iel, ieg, ied,
             gba, gbb, gbl, gbg, gbd,
             out_ref):
    it = items_ref[...]
    onehot = (it == lax.broadcasted_iota(jnp.int32, (TBLK, NI), 1)
              ).astype(jnp.float32)

    def ig(r):
        return jnp.dot(onehot, r[...], preferred_element_type=jnp.float32)

    def coef(gb, ub, ib, ue, ie):
        x = ue[...]
        rows = jnp.stack([x[:, L * v:L * (v + 1)] for v in range(8)],
                         axis=1).reshape(TBLK, L)
        d = jnp.sum(rows * ig(ie), axis=1, keepdims=True)
        return gb[0, 0] + ub[...] + ig(ib) + d

    alpha = coef(gba, uba, iba, gea, iea)
    beta = coef(gbb, ubb, ibb, geb, ieb)
    lamda = coef(gbl, ubl, ibl, gel, iel)
    gamma = coef(gbg, ubg, ibg, geg, ieg)
    delta = coef(gbd, ubd, ibd, ged, ied)

    dist = ig(dist_ref)
    price = ig(price_ref)

    rating = lax.broadcasted_iota(jnp.int32, (TBLK, 5), 1).astype(jnp.float32) + 1.0
    x = jnp.tanh(rating - refr[...])
    x_pos = (x > 0).astype(jnp.float32)
    x_neg = 1.0 - x_pos
    x_ = price * jnp.abs(x)
    v = x_ ** (alpha * x_pos + beta * x_neg)
    value = v * (x_pos - lamda * x_neg)
    w_exp = x_pos * gamma + x_neg * delta
    w_nom = dist ** w_exp
    w_den = (w_nom + (1.0 - dist) ** w_exp) ** (1.0 / w_exp)
    out_ref[...] = jnp.sum((w_nom / w_den) * value, axis=1, keepdims=True)


def _tc_math(items, ge, gus, p):
    col = lambda a: pl.BlockSpec((TBLK, 1), lambda i: (i, 0))
    full = lambda a: pl.BlockSpec(a.shape, lambda i: (0,) * a.ndim)
    emb = pl.BlockSpec((TBLK // 8, 128), lambda i: (i, 0))

    items2 = items.reshape(BATCH, 1)
    gus2 = [g.reshape(BATCH, 1) for g in gus]
    dist = p["dist"]
    price2 = p["price"].reshape(NI, 1)
    ibs = [p["ib_" + t] for t in ("a", "b", "l", "g", "d")]
    ies = [p["ie_" + t] for t in ("a", "b", "l", "g", "d")]
    gbs = [p["gb_" + t] for t in ("a", "b", "l", "g", "d")]

    args = [items2] + list(ge) + gus2 + [dist, price2] + ibs + ies + gbs
    specs = ([col(items2)] + [emb] * 5 + [col(g) for g in gus2]
             + [full(dist), full(price2)]
             + [full(a) for a in ibs] + [full(a) for a in ies]
             + [full(a) for a in gbs])
    out = pl.pallas_call(
        _tc_body,
        grid=(NTB,),
        in_specs=specs,
        out_specs=pl.BlockSpec((TBLK, 1), lambda i: (i, 0)),
        out_shape=jax.ShapeDtypeStruct((BATCH, 1), jnp.float32),
    )(*args)
    return out.reshape(BATCH)


def kernel(params, users, items):
    p = params
    U = p["ref"].shape[0]
    ue_tabs = [p["ue_" + t] for t in ("a", "b", "l", "g", "d")]
    us_tabs = [p["ub_" + t].reshape(U) for t in ("a", "b", "l", "g", "d")]
    us_tabs.append(p["ref"].reshape(U))
    lin = _relayout(ue_tabs)
    g = _sc_gather(users, lin, us_tabs)
    ge, gus = g[0:5], g[5:11]
    return _tc_math(items, ge, gus, p)


# speedup vs baseline: 4.8007x; 4.1589x over previous
import functools

import jax
import jax.numpy as jnp
from jax import lax
from jax.experimental import pallas as pl
from jax.experimental.pallas import tpu as pltpu
from jax.experimental.pallas import tpu_sc as plsc

BATCH = 16384
L = 16
NI = 100
NC = 2
NS = 16
NW = NC * NS
BPW = BATCH // NW
CH = 128
NCH = BPW // CH

RB = 10240
TBLK = 2048
NTB = BATCH // TBLK


def _relayout_body(*refs):
    parts = [refs[i][...] for i in range(5)]
    parts += [refs[5 + i][...] for i in range(6)]
    parts.append(jnp.zeros((128 - 5 * L - 6, RB), jnp.float32))
    x = jnp.concatenate(parts, axis=0)
    refs[11][...] = jnp.transpose(x, (1, 0))


def _relayout(ue_tabs, us_tabs):
    U = ue_tabs[0].shape[0]
    n = (U + RB - 1) // RB
    ins = [t.T for t in ue_tabs] + [t.T for t in us_tabs]
    in_specs = ([pl.BlockSpec((L, RB), lambda i: (0, i))] * 5
                + [pl.BlockSpec((1, RB), lambda i: (0, i))] * 6)
    out = pl.pallas_call(
        _relayout_body,
        grid=(n,),
        in_specs=in_specs,
        out_specs=pl.BlockSpec((RB, 128), lambda i: (i, 0)),
        out_shape=jax.ShapeDtypeStruct((U, 128), jnp.float32),
    )(*ins)
    return out


def _sc_gather(users, table):
    mesh = plsc.VectorSubcoreMesh(core_axis_name="c", subcore_axis_name="s")
    out_type = jax.ShapeDtypeStruct((BATCH, 128), jnp.float32)
    scratch = ([pltpu.VMEM((NCH, CH), jnp.int32)]
               + [pltpu.VMEM((CH, 128), jnp.float32)] * NCH
               + [pltpu.SemaphoreType.DMA])

    @functools.partial(pl.kernel, mesh=mesh, out_type=out_type,
                       scratch_types=scratch,
                       compiler_params=pltpu.CompilerParams(
                           use_tc_tiling_on_sc=False))
    def k(users_hbm, tab, out, idx_v, *rest):
        bufs = rest[0:NCH]
        sem = rest[NCH]
        wid = lax.axis_index("s") * NC + lax.axis_index("c")
        base = wid * BPW
        for c in range(NCH):
            pltpu.sync_copy(users_hbm.at[pl.ds(base + c * CH, CH)],
                            idx_v.at[c])
        cps = [pltpu.async_copy(tab.at[idx_v.at[c]], bufs[c], sem)
               for c in range(NCH)]
        for c in range(NCH):
            cps[c].wait()
            pltpu.sync_copy(bufs[c], out.at[pl.ds(base + c * CH, CH), :])

    return k(users, table)


def _tc_body(items_ref, rows_ref,
             dist_ref, price_ref,
             iba, ibb, ibl, ibg, ibd,
             iea, ieb, iel, ieg, ied,
             gba, gbb, gbl, gbg, gbd,
             out_ref):
    it = items_ref[...]
    rows = rows_ref[...]
    onehot = (it == lax.broadcasted_iota(jnp.int32, (TBLK, NI), 1)
              ).astype(jnp.float32)

    def ig(r):
        return jnp.dot(onehot, r[...], preferred_element_type=jnp.float32)

    def coef(i, gb, ib, ie):
        ue = rows[:, L * i:L * (i + 1)]
        ub = rows[:, 80 + i:81 + i]
        d = jnp.sum(ue * ig(ie), axis=1, keepdims=True)
        return gb[0, 0] + ub + ig(ib) + d

    alpha = coef(0, gba, iba, iea)
    beta = coef(1, gbb, ibb, ieb)
    lamda = coef(2, gbl, ibl, iel)
    gamma = coef(3, gbg, ibg, ieg)
    delta = coef(4, gbd, ibd, ied)
    refr = rows[:, 85:86]

    dist = ig(dist_ref)
    price = ig(price_ref)

    rating = lax.broadcasted_iota(jnp.int32, (TBLK, 5), 1).astype(jnp.float32) + 1.0
    x = jnp.tanh(rating - refr)
    x_pos = (x > 0).astype(jnp.float32)
    x_neg = 1.0 - x_pos
    x_ = price * jnp.abs(x)
    v = x_ ** (alpha * x_pos + beta * x_neg)
    value = v * (x_pos - lamda * x_neg)
    w_exp = x_pos * gamma + x_neg * delta
    w_nom = dist ** w_exp
    w_den = (w_nom + (1.0 - dist) ** w_exp) ** (1.0 / w_exp)
    out_ref[...] = jnp.sum((w_nom / w_den) * value, axis=1, keepdims=True)


def _tc_math(items, rows, p):
    full = lambda a: pl.BlockSpec(a.shape, lambda i: (0,) * a.ndim)

    items2 = items.reshape(BATCH, 1)
    dist = p["dist"]
    price2 = p["price"].reshape(NI, 1)
    ibs = [p["ib_" + t] for t in ("a", "b", "l", "g", "d")]
    ies = [p["ie_" + t] for t in ("a", "b", "l", "g", "d")]
    gbs = [p["gb_" + t] for t in ("a", "b", "l", "g", "d")]

    args = [items2, rows, dist, price2] + ibs + ies + gbs
    specs = ([pl.BlockSpec((TBLK, 1), lambda i: (i, 0)),
              pl.BlockSpec((TBLK, 128), lambda i: (i, 0)),
              full(dist), full(price2)]
             + [full(a) for a in ibs] + [full(a) for a in ies]
             + [full(a) for a in gbs])
    out = pl.pallas_call(
        _tc_body,
        grid=(NTB,),
        in_specs=specs,
        out_specs=pl.BlockSpec((TBLK, 1), lambda i: (i, 0)),
        out_shape=jax.ShapeDtypeStruct((BATCH, 1), jnp.float32),
    )(*args)
    return out.reshape(BATCH)


def kernel(params, users, items):
    p = params
    ue_tabs = [p["ue_" + t] for t in ("a", "b", "l", "g", "d")]
    us_tabs = [p["ub_" + t] for t in ("a", "b", "l", "g", "d")]
    us_tabs.append(p["ref"])
    table = _relayout(ue_tabs, us_tabs)
    rows = _sc_gather(users, table)
    return _tc_math(items, rows, p)
